# baseline (speedup 1.0000x reference)
import functools

import jax
import jax.numpy as jnp
from jax import lax
from jax.experimental import pallas as pl
from jax.experimental.pallas import tpu as pltpu
from jax.experimental.pallas import tpu_sc as plsc
import numpy as np

MIN = float(np.finfo(np.float32).min)
L = 128
NC = 48
E = 96
NN = 32
RR = 2 * E + 2 * NC
NB = RR // 2
D = 512
NW = 32
NROW = E * L
ROWS_PER_W = NROW // NW
CH = 96


def _col(v_row, n):
    eye = (lax.broadcasted_iota(jnp.int32, (n, n), 0)
           == lax.broadcasted_iota(jnp.int32, (n, n), 1))
    return jnp.sum(jnp.where(eye, v_row, 0), axis=1, keepdims=True)


def _extract_body(ids_ref, lf_ref, lm_ref, ei_ref, nm_ref, em_ref,
                  mcol_ref, sb_ref, mapped_ref, fmask_ref):
    ids_f = ids_ref[...].astype(jnp.float32)
    lf_row = jnp.clip(lf_ref[0:1, :], 8, L)
    lm_row = jnp.minimum(jnp.maximum(lm_ref[0:1, :], 1), lf_row)
    e0_row = ei_ref[0:1, :]
    c_row = ei_ref[1:2, :]
    em_row = em_ref[0:1, :]
    nm_row = nm_ref[0:1, :]

    c_col = _col(c_row, E)
    e0_col = _col(e0_row, E)
    em_col = _col(em_row, E)
    io96r = lax.broadcasted_iota(jnp.int32, (E, E), 1)
    io96c = lax.broadcasted_iota(jnp.int32, (E, E), 0)
    before = (c_row < c_col) | ((c_row == c_col) & (io96r < io96c))
    p_col = jnp.sum(before.astype(jnp.int32), axis=1, keepdims=True)
    io48c96 = lax.broadcasted_iota(jnp.int32, (NC, E), 0)
    cum_incl = jnp.sum((c_row <= io48c96).astype(jnp.int32), axis=1,
                       keepdims=True)
    io48col = lax.broadcasted_iota(jnp.int32, (NC, 1), 0)

    row1_col = 2 * (p_col + c_col)
    rowC_col = 2 * (cum_incl + io48col)

    io288r_e = lax.broadcasted_iota(jnp.int32, (E, RR), 1)
    io288r_c = lax.broadcasted_iota(jnp.int32, (NC, RR), 1)
    oh1 = (row1_col == io288r_e)
    oh2 = (row1_col + 1 == io288r_e)
    ohC = (rowC_col == io288r_c)
    ohP = (rowC_col + 1 == io288r_c)

    def scat_e(mask, val_col):
        return jnp.sum(jnp.where(mask, val_col, 0), axis=0, keepdims=True)

    src_row = (scat_e(oh1, e0_col) + scat_e(oh2, em_col + NC)
               + scat_e(ohC, io48col))
    is_cen_row = scat_e(ohC, jnp.ones((NC, 1), jnp.int32))
    is_pad_row = scat_e(ohP, jnp.ones((NC, 1), jnp.int32))

    src_col = _col(src_row, RR)
    is_cen_col = _col(is_cen_row, RR)
    is_pad_col = _col(is_pad_row, RR)

    io48r_rr = lax.broadcasted_iota(jnp.int32, (RR, NC), 1)
    nm_g = jnp.sum(jnp.where(src_col == io48r_rr, nm_row, 0), axis=1,
                   keepdims=True)
    g_col = jnp.where(src_col < NC, nm_g, src_col - (NC - NN))

    io96r_rr = lax.broadcasted_iota(jnp.int32, (RR, E), 1)
    gm = (g_col == io96r_rr)
    lf_g = jnp.sum(jnp.where(gm, lf_row, 0), axis=1, keepdims=True)
    lm_g = jnp.sum(jnp.where(gm, lm_row, 0), axis=1, keepdims=True)
    len_col = jnp.where(is_pad_col > 0, 0,
                        jnp.where(is_cen_col > 0, lf_g, lm_g))

    io128 = lax.broadcasted_iota(jnp.int32, (RR, L), 1)
    mcol_ref[...] = jnp.where(
        io128 < len_col, jnp.where(is_cen_col > 0, io128, 0), L)

    io48r96 = lax.broadcasted_iota(jnp.int32, (E, NC), 1)
    nm_e0 = jnp.sum(jnp.where(e0_col == io48r96, nm_row, 0), axis=1,
                    keepdims=True)
    nm_ce = jnp.sum(jnp.where(c_col == io48r96, nm_row, 0), axis=1,
                    keepdims=True)
    nm_col = _col(nm_row, NC)

    blk_e_col = p_col + c_col
    blk_c_col = cum_incl + io48col
    io144_e = lax.broadcasted_iota(jnp.int32, (E, NB), 1)
    io144_c = lax.broadcasted_iota(jnp.int32, (NC, NB), 1)
    ohbe = (blk_e_col == io144_e)
    ohbc = (blk_c_col == io144_c)

    def sce(val_col):
        return jnp.sum(jnp.where(ohbe, val_col, 0), axis=0, keepdims=True)

    def scc(val_col):
        return jnp.sum(jnp.where(ohbc, val_col, 0), axis=0, keepdims=True)

    corB = sce(c_col) + scc(io48col)
    lastB = scc(jnp.ones((NC, 1), jnp.int32))
    g1B = sce(nm_e0) + scc(nm_col)
    g2B = sce(em_col + NN)
    gqB = sce(nm_ce) + scc(nm_col)
    sb_ref[...] = jnp.concatenate([corB, lastB, g1B, g2B, gqB], axis=0)

    io32r = lax.broadcasted_iota(jnp.int32, (NC, NN), 1)
    ohnm = (nm_col == io32r).astype(jnp.float32)
    mapped_ref[...] = lax.dot_general(
        ohnm, ids_f[:NN, :], (((1,), (0,)), ((), ())),
        preferred_element_type=jnp.float32).astype(jnp.int32)
    io96r_nc = lax.broadcasted_iota(jnp.int32, (NC, E), 1)
    nmm = (nm_col == io96r_nc)
    lm_nm = jnp.sum(jnp.where(nmm, lm_row, 0), axis=1, keepdims=True)
    lf_nm = jnp.sum(jnp.where(nmm, lf_row, 0), axis=1, keepdims=True)
    io128_nc = lax.broadcasted_iota(jnp.int32, (NC, L), 1)
    fmask_ref[...] = ((io128_nc >= lm_nm) & (io128_nc < lf_nm)).astype(jnp.int32)


def _extract(ids, lf, lm, ei, nm, em):
    return pl.pallas_call(
        _extract_body,
        out_shape=[
            jax.ShapeDtypeStruct((RR, L), jnp.int32),
            jax.ShapeDtypeStruct((5, NB), jnp.int32),
            jax.ShapeDtypeStruct((NC, L), jnp.int32),
            jax.ShapeDtypeStruct((NC, L), jnp.int32),
        ],
    )(ids, lf.reshape(1, E), lm.reshape(1, E), ei, nm.reshape(1, NC),
      em.reshape(1, E))


def _gather_body(idx_hbm, table_hbm, out_hbm, idx_v, rows_a, rows_b, sem_a,
                 sem_b):
    wid = lax.axis_index("s") * 2 + lax.axis_index("c")
    base = wid * ROWS_PER_W
    nch = ROWS_PER_W // CH
    pltpu.sync_copy(idx_hbm.at[pl.ds(base, ROWS_PER_W)], idx_v)
    bufs = [rows_a, rows_b]
    sems = [sem_a, sem_b]
    cps = [None, None]
    cps[0] = pltpu.async_copy(table_hbm.at[idx_v.at[pl.ds(0, CH)]], rows_a,
                              sem_a)
    for i in range(nch):
        if i + 1 < nch:
            cps[(i + 1) % 2] = pltpu.async_copy(
                table_hbm.at[idx_v.at[pl.ds((i + 1) * CH, CH)]],
                bufs[(i + 1) % 2], sems[(i + 1) % 2])
        cps[i % 2].wait()
        pltpu.sync_copy(bufs[i % 2], out_hbm.at[pl.ds(base + i * CH, CH)])


def _pack_table(W_emb):
    bits = lax.bitcast_convert_type(W_emb, jnp.uint32)
    rne = bits + jnp.uint32(0x7FFF) + ((bits >> 16) & jnp.uint32(1))
    hi = rne >> 16
    packed = hi[:, : D // 2] | (hi[:, D // 2:] << 16)
    return lax.bitcast_convert_type(packed, jnp.int32)


def _gather(ids_flat, W_packed):
    gk = functools.partial(
        pl.kernel,
        out_type=jax.ShapeDtypeStruct((NROW, D // 2), jnp.int32),
        mesh=plsc.VectorSubcoreMesh(core_axis_name="c", subcore_axis_name="s"),
        scratch_types=[
            pltpu.VMEM((ROWS_PER_W,), jnp.int32),
            pltpu.VMEM((CH, D // 2), jnp.int32),
            pltpu.VMEM((CH, D // 2), jnp.int32),
            pltpu.SemaphoreType.DMA,
            pltpu.SemaphoreType.DMA,
        ],
    )(_gather_body)
    return gk(ids_flat, W_packed)


def _unpack(k32):
    lo = lax.bitcast_convert_type(k32 << 16, jnp.float32)
    hi = lax.bitcast_convert_type(
        k32 & jnp.int32(np.int32(np.uint32(0xFFFF0000).view(np.int32))),
        jnp.float32)
    return jnp.concatenate(
        [lo.astype(jnp.bfloat16), hi.astype(jnp.bfloat16)], axis=1)


def _attn_body(cor_ref, last_ref, g1_ref, g2_ref, gq_ref,
               kbase_ref, mc_ref, wout_ref, o_ref, sA, sB, mstat, lstat, acc):
    b = pl.program_id(0)
    par = lax.rem(b, 2)
    even = par == 0
    inv = 1.0 / float(np.sqrt(D))

    s_even = sA[...]
    s_odd = sB[...]

    ba = jnp.minimum(b, NB - 1)
    q = _unpack(kbase_ref[gq_ref[ba]])
    k1 = _unpack(kbase_ref[g1_ref[ba]])
    k2 = _unpack(kbase_ref[g2_ref[ba]])
    kk = jnp.concatenate([k1, k2], axis=0)
    s = lax.dot_general(q, kk, (((1,), (1,)), ((), ())),
                        preferred_element_type=jnp.float32) * inv
    sA[...] = jnp.where(even, s, s_even)
    sB[...] = jnp.where(even, s_odd, s)

    valid = b > 0
    j = jnp.maximum(b - 1, 0)
    cb = cor_ref[j]
    first = (j == 0) | (cb != cor_ref[jnp.maximum(j - 1, 0)])
    kj1 = _unpack(kbase_ref[g1_ref[j]])
    kj2 = _unpack(kbase_ref[g2_ref[j]])
    kkj = jnp.concatenate([kj1, kj2], axis=0)
    mc = mc_ref[j]
    mcat = jnp.concatenate([mc[0:1, :], mc[1:2, :]], axis=1)
    qio = lax.broadcasted_iota(jnp.int32, (L, 2 * L), 0)
    sj = jnp.where(valid & (mcat <= qio), jnp.where(even, s_odd, s_even), MIN)
    m_prev = jnp.where(first, MIN, mstat[...])
    l_prev = jnp.where(first, 0.0, lstat[...])
    a_prev = jnp.where(first, 0.0, acc[...])
    m_new = jnp.maximum(m_prev, jnp.max(sj, axis=1, keepdims=True))
    alpha = jnp.exp(m_prev - m_new)
    p = jnp.where(valid, jnp.exp(sj - m_new), 0.0)
    l_new = alpha * l_prev + jnp.sum(p, axis=1, keepdims=True)
    a_new = (alpha * a_prev
             + lax.dot_general(p.astype(jnp.bfloat16), kkj,
                               (((1,), (0,)), ((), ())),
                               preferred_element_type=jnp.float32))
    mstat[...] = m_new
    lstat[...] = l_new
    acc[...] = a_new
    o_ref[cb] = lax.dot_general((a_new * (1.0 / l_new)).astype(jnp.bfloat16),
                                wout_ref[...].astype(jnp.bfloat16),
                                (((1,), (0,)), ((), ())),
                                preferred_element_type=jnp.float32)


def _attn(Kbase, mcol3, W_out, sb):
    prev = lambda b: jnp.maximum(b - 1, 0)
    grid_spec = pltpu.PrefetchScalarGridSpec(
        num_scalar_prefetch=5,
        grid=(NB + 1,),
        in_specs=[
            pl.BlockSpec((E, L, D // 2), lambda b, *_: (0, 0, 0)),
            pl.BlockSpec((NB, 2, L), lambda b, *_: (0, 0, 0)),
            pl.BlockSpec((D, D), lambda b, *_: (0, 0)),
        ],
        out_specs=pl.BlockSpec((NC, L, D), lambda b, *_: (0, 0, 0)),
        scratch_shapes=[
            pltpu.VMEM((L, 2 * L), jnp.float32),
            pltpu.VMEM((L, 2 * L), jnp.float32),
            pltpu.VMEM((L, 1), jnp.float32),
            pltpu.VMEM((L, 1), jnp.float32),
            pltpu.VMEM((L, D), jnp.float32),
        ],
    )
    return pl.pallas_call(
        _attn_body,
        grid_spec=grid_spec,
        out_shape=jax.ShapeDtypeStruct((NC, L, D), jnp.float32),
    )(sb[0], sb[1], sb[2], sb[3], sb[4], Kbase, mcol3, W_out)


def kernel(input_ids, len_full, len_masked, edge_index, node_map, edge_map,
           W_emb, W_out):
    ids = input_ids.astype(jnp.int32)
    mcol, sb, mapped_ids, fmask = _extract(
        ids, len_full.astype(jnp.int32), len_masked.astype(jnp.int32),
        edge_index.astype(jnp.int32), node_map.astype(jnp.int32),
        edge_map.astype(jnp.int32))
    Kflat = _gather(ids.reshape(NROW), _pack_table(W_emb))
    Kbase = Kflat.reshape(E, L, D // 2)
    mcol3 = mcol.reshape(NB, 2, L)
    logits = _attn(Kbase, mcol3, W_out, sb)
    return (logits[:, :-1, :], mapped_ids[:, 1:], fmask[:, 1:] != 0)


# speedup vs baseline: 1.1922x; 1.1922x over previous
import functools

import jax
import jax.numpy as jnp
from jax import lax
from jax.experimental import pallas as pl
from jax.experimental.pallas import tpu as pltpu
from jax.experimental.pallas import tpu_sc as plsc
import numpy as np

MIN = float(np.finfo(np.float32).min)
L = 128
NC = 48
E = 96
NN = 32
RPB = 4
RR = RPB * E
NB = RR // RPB
D = 512
NW = 32
NROW = E * L
ROWS_PER_W = NROW // NW
CH = 96


def _col(v_row, n):
    eye = (lax.broadcasted_iota(jnp.int32, (n, n), 0)
           == lax.broadcasted_iota(jnp.int32, (n, n), 1))
    return jnp.sum(jnp.where(eye, v_row, 0), axis=1, keepdims=True)


def _extract_body(ids_ref, lf_ref, lm_ref, ei_ref, nm_ref, em_ref,
                  mcol_ref, sb_ref, mapped_ref, fmask_ref):
    ids_f = ids_ref[...].astype(jnp.float32)
    lf_row = jnp.clip(lf_ref[0:1, :], 8, L)
    lm_row = jnp.minimum(jnp.maximum(lm_ref[0:1, :], 1), lf_row)
    e0_row = ei_ref[0:1, :]
    c_row = ei_ref[1:2, :]
    em_row = em_ref[0:1, :]
    nm_row = nm_ref[0:1, :]

    c_col = _col(c_row, E)
    e0_col = _col(e0_row, E)
    em_col = _col(em_row, E)
    io96r = lax.broadcasted_iota(jnp.int32, (E, E), 1)
    io96c = lax.broadcasted_iota(jnp.int32, (E, E), 0)
    before = (c_row < c_col) | ((c_row == c_col) & (io96r < io96c))
    p_col = jnp.sum(before.astype(jnp.int32), axis=1, keepdims=True)
    io48c96 = lax.broadcasted_iota(jnp.int32, (NC, E), 0)
    io48col = lax.broadcasted_iota(jnp.int32, (NC, 1), 0)
    d_col = jnp.sum((c_row == io48c96).astype(jnp.int32), axis=1,
                    keepdims=True)
    nb_col = jnp.right_shift(d_col, 1) + 1
    eye48 = (lax.broadcasted_iota(jnp.int32, (NC, NC), 0)
             == lax.broadcasted_iota(jnp.int32, (NC, NC), 1))
    nb_row = jnp.sum(jnp.where(eye48, nb_col, 0), axis=0, keepdims=True)
    io48_0 = lax.broadcasted_iota(jnp.int32, (NC, NC), 0)
    io48_1 = lax.broadcasted_iota(jnp.int32, (NC, NC), 1)
    cumnb_col = jnp.sum(jnp.where(io48_1 < io48_0, nb_row, 0), axis=1,
                        keepdims=True)
    io48r96e = lax.broadcasted_iota(jnp.int32, (E, NC), 1)
    cumnb_e = jnp.sum(jnp.where(io48r96e < c_col, nb_row, 0), axis=1,
                      keepdims=True)
    ce_col = jnp.sum((c_row < c_col).astype(jnp.int32), axis=1,
                     keepdims=True)
    pp_col = p_col - ce_col

    row1_col = RPB * cumnb_e + 2 * pp_col
    rowC_col = RPB * cumnb_col + 2 * d_col

    io288r_e = lax.broadcasted_iota(jnp.int32, (E, RR), 1)
    io288r_c = lax.broadcasted_iota(jnp.int32, (NC, RR), 1)
    oh1 = (row1_col == io288r_e)
    oh2 = (row1_col + 1 == io288r_e)
    ohC = (rowC_col == io288r_c)

    def scat_e(mask, val_col):
        return jnp.sum(jnp.where(mask, val_col, 0), axis=0, keepdims=True)

    src_row = (scat_e(oh1, e0_col) + scat_e(oh2, em_col + NC)
               + scat_e(ohC, io48col))
    is_cen_row = scat_e(ohC, jnp.ones((NC, 1), jnp.int32))
    is_run_row = (scat_e(oh1, jnp.ones((E, 1), jnp.int32))
                  + scat_e(oh2, jnp.ones((E, 1), jnp.int32)) + is_cen_row)

    src_col = _col(src_row, RR)
    is_cen_col = _col(is_cen_row, RR)
    is_run_col = _col(is_run_row, RR)

    io48r_rr = lax.broadcasted_iota(jnp.int32, (RR, NC), 1)
    nm_g = jnp.sum(jnp.where(src_col == io48r_rr, nm_row, 0), axis=1,
                   keepdims=True)
    g_col = jnp.where(src_col < NC, nm_g, src_col - (NC - NN))

    io96r_rr = lax.broadcasted_iota(jnp.int32, (RR, E), 1)
    gm = (g_col == io96r_rr)
    lf_g = jnp.sum(jnp.where(gm, lf_row, 0), axis=1, keepdims=True)
    lm_g = jnp.sum(jnp.where(gm, lm_row, 0), axis=1, keepdims=True)
    len_col = jnp.where(is_run_col > 0,
                        jnp.where(is_cen_col > 0, lf_g, lm_g), 0)

    io128 = lax.broadcasted_iota(jnp.int32, (RR, L), 1)
    mcol_ref[...] = jnp.where(
        io128 < len_col, jnp.where(is_cen_col > 0, io128, 0), L)

    nm_col = _col(nm_row, NC)

    ioblk = lax.broadcasted_iota(jnp.int32, (NC, NB), 1)
    memb = (cumnb_col <= ioblk) & (ioblk < cumnb_col + nb_col)

    def scc(val_col):
        return jnp.sum(jnp.where(memb, val_col, 0), axis=0, keepdims=True)

    cov = jnp.sum(memb.astype(jnp.int32), axis=0, keepdims=True)
    corB = scc(io48col) + (NC - 1) * (1 - cov)
    lastB = jnp.sum((ioblk == cumnb_col + nb_col - 1).astype(jnp.int32),
                    axis=0, keepdims=True)
    gqB = scc(nm_col)
    io384_0 = lax.broadcasted_iota(jnp.int32, (RR, NB), 0)
    io384_1 = lax.broadcasted_iota(jnp.int32, (RR, NB), 1)
    gks = []
    for k in range(RPB):
        mk = (io384_0 == RPB * io384_1 + k)
        gks.append(jnp.sum(jnp.where(mk, g_col, 0), axis=0, keepdims=True))
    sb_ref[...] = jnp.concatenate([corB, lastB] + gks + [gqB], axis=0)

    io32r = lax.broadcasted_iota(jnp.int32, (NC, NN), 1)
    ohnm = (nm_col == io32r).astype(jnp.float32)
    mapped_ref[...] = lax.dot_general(
        ohnm, ids_f[:NN, :], (((1,), (0,)), ((), ())),
        preferred_element_type=jnp.float32).astype(jnp.int32)
    io96r_nc = lax.broadcasted_iota(jnp.int32, (NC, E), 1)
    nmm = (nm_col == io96r_nc)
    lm_nm = jnp.sum(jnp.where(nmm, lm_row, 0), axis=1, keepdims=True)
    lf_nm = jnp.sum(jnp.where(nmm, lf_row, 0), axis=1, keepdims=True)
    io128_nc = lax.broadcasted_iota(jnp.int32, (NC, L), 1)
    fmask_ref[...] = ((io128_nc >= lm_nm) & (io128_nc < lf_nm)).astype(jnp.int32)


def _extract(ids, lf, lm, ei, nm, em):
    return pl.pallas_call(
        _extract_body,
        out_shape=[
            jax.ShapeDtypeStruct((RR, L), jnp.int32),
            jax.ShapeDtypeStruct((2 + RPB + 1, NB), jnp.int32),
            jax.ShapeDtypeStruct((NC, L), jnp.int32),
            jax.ShapeDtypeStruct((NC, L), jnp.int32),
        ],
    )(ids, lf.reshape(1, E), lm.reshape(1, E), ei, nm.reshape(1, NC),
      em.reshape(1, E))


def _gather_body(idx_hbm, table_hbm, out_hbm, idx_v, rows_a, rows_b, sem_a,
                 sem_b):
    wid = lax.axis_index("s") * 2 + lax.axis_index("c")
    base = wid * ROWS_PER_W
    nch = ROWS_PER_W // CH
    pltpu.sync_copy(idx_hbm.at[pl.ds(base, ROWS_PER_W)], idx_v)
    bufs = [rows_a, rows_b]
    sems = [sem_a, sem_b]
    cps = [None, None]
    cps[0] = pltpu.async_copy(table_hbm.at[idx_v.at[pl.ds(0, CH)]], rows_a,
                              sem_a)
    for i in range(nch):
        if i + 1 < nch:
            cps[(i + 1) % 2] = pltpu.async_copy(
                table_hbm.at[idx_v.at[pl.ds((i + 1) * CH, CH)]],
                bufs[(i + 1) % 2], sems[(i + 1) % 2])
        cps[i % 2].wait()
        pltpu.sync_copy(bufs[i % 2], out_hbm.at[pl.ds(base + i * CH, CH)])


def _pack_table(W_emb):
    bits = lax.bitcast_convert_type(W_emb, jnp.uint32)
    rne = bits + jnp.uint32(0x7FFF) + ((bits >> 16) & jnp.uint32(1))
    hi = rne >> 16
    packed = hi[:, : D // 2] | (hi[:, D // 2:] << 16)
    return lax.bitcast_convert_type(packed, jnp.int32)


def _gather(ids_flat, W_packed):
    gk = functools.partial(
        pl.kernel,
        out_type=jax.ShapeDtypeStruct((NROW, D // 2), jnp.int32),
        mesh=plsc.VectorSubcoreMesh(core_axis_name="c", subcore_axis_name="s"),
        scratch_types=[
            pltpu.VMEM((ROWS_PER_W,), jnp.int32),
            pltpu.VMEM((CH, D // 2), jnp.int32),
            pltpu.VMEM((CH, D // 2), jnp.int32),
            pltpu.SemaphoreType.DMA,
            pltpu.SemaphoreType.DMA,
        ],
    )(_gather_body)
    return gk(ids_flat, W_packed)


def _unpack(k32):
    lo = lax.bitcast_convert_type(k32 << 16, jnp.float32)
    hi = lax.bitcast_convert_type(
        k32 & jnp.int32(np.int32(np.uint32(0xFFFF0000).view(np.int32))),
        jnp.float32)
    return jnp.concatenate(
        [lo.astype(jnp.bfloat16), hi.astype(jnp.bfloat16)], axis=1)


def _attn_body(sb_ref, kbase_ref, mc_ref, wout_ref, o_ref,
               sA, sB, mstat, lstat, acc):
    inv = 1.0 / float(np.sqrt(D))

    def step(b, carry):
        _attn_step(b, sb_ref, kbase_ref, mc_ref, wout_ref, o_ref,
                   sA, sB, mstat, lstat, acc, inv)
        return carry

    lax.fori_loop(0, NB + 1, step, jnp.int32(0))


def _attn_step(b, sb_ref, kbase_ref, mc_ref, wout_ref, o_ref,
               sA, sB, mstat, lstat, acc, inv):
    par = lax.rem(b, 2)
    even = par == 0

    s_even = sA[...]
    s_odd = sB[...]

    ba = jnp.minimum(b, NB - 1)
    q = _unpack(kbase_ref[sb_ref[6, ba]])
    kk = jnp.concatenate(
        [_unpack(kbase_ref[sb_ref[2, ba]]), _unpack(kbase_ref[sb_ref[3, ba]]),
         _unpack(kbase_ref[sb_ref[4, ba]]), _unpack(kbase_ref[sb_ref[5, ba]])],
        axis=0)
    s = lax.dot_general(q, kk, (((1,), (1,)), ((), ())),
                        preferred_element_type=jnp.float32) * inv
    sA[...] = jnp.where(even, s, s_even)
    sB[...] = jnp.where(even, s_odd, s)

    valid = b > 0
    j = jnp.maximum(b - 1, 0)
    cb = sb_ref[0, j]
    first = (j == 0) | (cb != sb_ref[0, jnp.maximum(j - 1, 0)])
    kkj = jnp.concatenate(
        [_unpack(kbase_ref[sb_ref[2, j]]), _unpack(kbase_ref[sb_ref[3, j]]),
         _unpack(kbase_ref[sb_ref[4, j]]), _unpack(kbase_ref[sb_ref[5, j]])],
        axis=0)
    mc = mc_ref[j]
    mcat = jnp.concatenate(
        [mc[0:1, :], mc[1:2, :], mc[2:3, :], mc[3:4, :]], axis=1)
    qio = lax.broadcasted_iota(jnp.int32, (L, RPB * L), 0)
    sj = jnp.where(valid & (mcat <= qio), jnp.where(even, s_odd, s_even), MIN)
    m_prev = jnp.where(first, MIN, mstat[...])
    l_prev = jnp.where(first, 0.0, lstat[...])
    a_prev = jnp.where(first, 0.0, acc[...])
    m_new = jnp.maximum(m_prev, jnp.max(sj, axis=1, keepdims=True))
    alpha = jnp.exp(m_prev - m_new)
    p = jnp.where(valid, jnp.exp(sj - m_new), 0.0)
    l_new = alpha * l_prev + jnp.sum(p, axis=1, keepdims=True)
    a_new = (alpha * a_prev
             + lax.dot_general(p.astype(jnp.bfloat16), kkj,
                               (((1,), (0,)), ((), ())),
                               preferred_element_type=jnp.float32))
    mstat[...] = m_new
    lstat[...] = l_new
    acc[...] = a_new
    o_ref[cb] = lax.dot_general((a_new * (1.0 / l_new)).astype(jnp.bfloat16),
                                wout_ref[...].astype(jnp.bfloat16),
                                (((1,), (0,)), ((), ())),
                                preferred_element_type=jnp.float32)


def _attn(Kbase, mcol3, W_out, sb):
    return pl.pallas_call(
        _attn_body,
        in_specs=[
            pl.BlockSpec(memory_space=pltpu.SMEM),
            pl.BlockSpec((E, L, D // 2), lambda: (0, 0, 0)),
            pl.BlockSpec((NB, RPB, L), lambda: (0, 0, 0)),
            pl.BlockSpec((D, D), lambda: (0, 0)),
        ],
        out_specs=pl.BlockSpec((NC, L, D), lambda: (0, 0, 0)),
        scratch_shapes=[
            pltpu.VMEM((L, RPB * L), jnp.float32),
            pltpu.VMEM((L, RPB * L), jnp.float32),
            pltpu.VMEM((L, 1), jnp.float32),
            pltpu.VMEM((L, 1), jnp.float32),
            pltpu.VMEM((L, D), jnp.float32),
        ],
        out_shape=jax.ShapeDtypeStruct((NC, L, D), jnp.float32),
    )(sb, Kbase, mcol3, W_out)


def kernel(input_ids, len_full, len_masked, edge_index, node_map, edge_map,
           W_emb, W_out):
    ids = input_ids.astype(jnp.int32)
    mcol, sb, mapped_ids, fmask = _extract(
        ids, len_full.astype(jnp.int32), len_masked.astype(jnp.int32),
        edge_index.astype(jnp.int32), node_map.astype(jnp.int32),
        edge_map.astype(jnp.int32))
    Kflat = _gather(ids.reshape(NROW), _pack_table(W_emb))
    Kbase = Kflat.reshape(E, L, D // 2)
    mcol3 = mcol.reshape(NB, RPB, L)
    logits = _attn(Kbase, mcol3, W_out, sb)
    return (logits[:, :-1, :], mapped_ids[:, 1:], fmask[:, 1:] != 0)
